# baseline (speedup 1.0000x reference)
import functools

import jax
import jax.numpy as jnp
from jax import lax
from jax.experimental import pallas as pl
from jax.experimental.pallas import tpu as pltpu
from jax.experimental.pallas import tpu_sc as plsc

N = 10000
D = 256
H = 128
E = 160000
N_PAD = 10240
E_PAD = 163840
G = 128
ROWS_PER_TILE = N_PAD // 16
EGROUPS = E_PAD // G
EG_PER_TILE = EGROUPS // 16
EG_PER_WORKER = EGROUPS // 32

_mesh = plsc.VectorSubcoreMesh(core_axis_name="c", subcore_axis_name="s")


@functools.partial(
    pl.kernel,
    out_type=(
        jax.ShapeDtypeStruct((N_PAD,), jnp.float32),
        jax.ShapeDtypeStruct((N_PAD,), jnp.float32),
    ),
    mesh=_mesh,
    scratch_types=[
        pltpu.VMEM((EG_PER_WORKER, G), jnp.int32),
        pltpu.VMEM((G,), jnp.float32),
        pltpu.VMEM((ROWS_PER_TILE,), jnp.float32),
        pltpu.VMEM_SHARED((N_PAD,), jnp.float32),
    ],
)
def _deg(col_hbm, out0, out1, cidx, ones, zbuf, hist):
    c = lax.axis_index("c")
    s = lax.axis_index("s")

    def fill(i, carry):
        ones[pl.ds(i * 16, 16)] = jnp.full((16,), 1.0, jnp.float32)
        return carry

    lax.fori_loop(0, G // 16, fill, 0)

    def zfill(i, carry):
        zbuf[pl.ds(i * 16, 16)] = jnp.zeros((16,), jnp.float32)
        return carry

    lax.fori_loop(0, ROWS_PER_TILE // 16, zfill, 0)
    pltpu.sync_copy(zbuf, hist.at[pl.ds(s * ROWS_PER_TILE, ROWS_PER_TILE)])
    plsc.subcore_barrier()

    base = c * (EGROUPS // 2) + s * EG_PER_WORKER
    pltpu.sync_copy(col_hbm.at[pl.ds(base, EG_PER_WORKER)], cidx)

    def body(j, carry):
        pltpu.sync_copy(ones, hist.at[cidx.at[j]], add=True)
        return carry

    lax.fori_loop(0, EG_PER_WORKER, body, 0)
    plsc.subcore_barrier()

    sl = pl.ds(s * ROWS_PER_TILE, ROWS_PER_TILE)

    @pl.when(c == 0)
    def _():
        pltpu.sync_copy(hist.at[sl], out0.at[sl])

    @pl.when(c == 1)
    def _():
        pltpu.sync_copy(hist.at[sl], out1.at[sl])


@functools.partial(
    pl.kernel,
    out_type=(
        jax.ShapeDtypeStruct((N_PAD, H), jnp.float32),
        jax.ShapeDtypeStruct((N_PAD, H), jnp.float32),
    ),
    mesh=_mesh,
    scratch_types=[
        pltpu.VMEM((EG_PER_TILE // 2, G), jnp.int32),
        pltpu.VMEM((EG_PER_TILE // 2, G), jnp.int32),
        pltpu.VMEM((G, H), jnp.float32),
        pltpu.VMEM((G, H), jnp.float32),
        pltpu.VMEM_SHARED((N_PAD, H), jnp.float32),
        pltpu.SemaphoreType.DMA,
        pltpu.SemaphoreType.DMA,
    ],
)
def _scatter(row_hbm, col_hbm, ya_hbm, yb_hbm, outa, outb, ridx, cidx,
             bufa, bufb, acc, sema, semb):
    c = lax.axis_index("c")
    s = lax.axis_index("s")

    def zrow(i, carry):
        for k in range(H // 16):
            bufa[i, pl.ds(k * 16, 16)] = jnp.zeros((16,), jnp.float32)
        return carry

    lax.fori_loop(0, G, zrow, 0)
    for t in range(ROWS_PER_TILE // G):
        pltpu.sync_copy(bufa, acc.at[pl.ds(s * ROWS_PER_TILE + t * G, G)])
    plsc.subcore_barrier()

    PH = EG_PER_TILE // 2

    def make_body(y_hbm):
        def gstart(j, buf, sem):
            jw = jnp.where(j < PH, j, 0)
            pltpu.async_copy(y_hbm.at[ridx.at[jw]], buf, sem)

        def gwait(buf, sem):
            pltpu.make_async_copy(y_hbm.at[ridx.at[0]], buf, sem).wait()

        def run():
            for phase in range(2):
                ebase = s * EG_PER_TILE + phase * PH
                pltpu.sync_copy(row_hbm.at[pl.ds(ebase, PH)], ridx)
                pltpu.sync_copy(col_hbm.at[pl.ds(ebase, PH)], cidx)
                gstart(jnp.int32(0), bufa, sema)
                gstart(jnp.int32(1), bufb, semb)

                def body(k, carry):
                    j = k * 2
                    gwait(bufa, sema)
                    pltpu.sync_copy(bufa, acc.at[cidx.at[j]], add=True)
                    gstart(j + 2, bufa, sema)
                    gwait(bufb, semb)
                    pltpu.sync_copy(bufb, acc.at[cidx.at[j + 1]], add=True)
                    gstart(j + 3, bufb, semb)
                    return carry

                lax.fori_loop(0, PH // 2, body, 0)
                gwait(bufa, sema)
                gwait(bufb, semb)

        return run

    @pl.when(c == 0)
    def _():
        make_body(ya_hbm)()

    @pl.when(c == 1)
    def _():
        make_body(yb_hbm)()

    plsc.subcore_barrier()

    sl = pl.ds(s * ROWS_PER_TILE, ROWS_PER_TILE)

    @pl.when(c == 0)
    def _():
        pltpu.sync_copy(acc.at[sl], outa.at[sl])

    @pl.when(c == 1)
    def _():
        pltpu.sync_copy(acc.at[sl], outb.at[sl])


_RB = 1024
_GRID = N_PAD // _RB


def _gelu(v):
    return 0.5 * v * (1.0 + lax.erf(v * 0.7071067811865476))


def _pre_body(x_ref, w_ref, dinv_ref, ya_ref, yb_ref):
    y = jnp.dot(x_ref[...], w_ref[...], preferred_element_type=jnp.float32)
    y = y * dinv_ref[...]
    ya_ref[...] = y[:, :H]
    yb_ref[...] = y[:, H:]


def _pre(x, w, dinv):
    return pl.pallas_call(
        _pre_body,
        grid=(_GRID,),
        in_specs=[
            pl.BlockSpec((_RB, D), lambda i: (i, 0)),
            pl.BlockSpec((D, D), lambda i: (0, 0)),
            pl.BlockSpec((_RB, 1), lambda i: (i, 0)),
        ],
        out_specs=(
            pl.BlockSpec((_RB, H), lambda i: (i, 0)),
            pl.BlockSpec((_RB, H), lambda i: (i, 0)),
        ),
        out_shape=(
            jax.ShapeDtypeStruct((N_PAD, H), jnp.float32),
            jax.ShapeDtypeStruct((N_PAD, H), jnp.float32),
        ),
    )(x, w, dinv)


def _mid_body(aa_ref, ab_ref, ya_ref, yb_ref, dinv_ref, b_ref, w_ref,
              oa_ref, ob_ref):
    dinv = dinv_ref[...]
    ha = dinv * (aa_ref[...] + 2.0 * ya_ref[...])
    hb = dinv * (ab_ref[...] + 2.0 * yb_ref[...])
    h = jnp.concatenate([ha, hb], axis=1) + b_ref[...]
    h = _gelu(h)
    y = jnp.dot(h, w_ref[...], preferred_element_type=jnp.float32) * dinv
    oa_ref[...] = y[:, :H]
    ob_ref[...] = y[:, H:]


def _mid(aa, ab, ya, yb, dinv, b, w):
    return pl.pallas_call(
        _mid_body,
        grid=(_GRID,),
        in_specs=[
            pl.BlockSpec((_RB, H), lambda i: (i, 0)),
            pl.BlockSpec((_RB, H), lambda i: (i, 0)),
            pl.BlockSpec((_RB, H), lambda i: (i, 0)),
            pl.BlockSpec((_RB, H), lambda i: (i, 0)),
            pl.BlockSpec((_RB, 1), lambda i: (i, 0)),
            pl.BlockSpec((1, D), lambda i: (0, 0)),
            pl.BlockSpec((D, D), lambda i: (0, 0)),
        ],
        out_specs=(
            pl.BlockSpec((_RB, H), lambda i: (i, 0)),
            pl.BlockSpec((_RB, H), lambda i: (i, 0)),
        ),
        out_shape=(
            jax.ShapeDtypeStruct((N_PAD, H), jnp.float32),
            jax.ShapeDtypeStruct((N_PAD, H), jnp.float32),
        ),
    )(aa, ab, ya, yb, dinv, b, w)


def _fin_body(aa_ref, ab_ref, ya_ref, yb_ref, dinv_ref, b_ref, w_ref,
              bp_ref, o_ref):
    dinv = dinv_ref[...]
    ha = dinv * (aa_ref[...] + 2.0 * ya_ref[...])
    hb = dinv * (ab_ref[...] + 2.0 * yb_ref[...])
    h = jnp.concatenate([ha, hb], axis=1) + b_ref[...]
    h = _gelu(h)
    o_ref[...] = (
        jnp.dot(h, w_ref[...], preferred_element_type=jnp.float32) + bp_ref[...]
    )


def _fin(aa, ab, ya, yb, dinv, b, w, bp):
    return pl.pallas_call(
        _fin_body,
        grid=(_GRID,),
        in_specs=[
            pl.BlockSpec((_RB, H), lambda i: (i, 0)),
            pl.BlockSpec((_RB, H), lambda i: (i, 0)),
            pl.BlockSpec((_RB, H), lambda i: (i, 0)),
            pl.BlockSpec((_RB, H), lambda i: (i, 0)),
            pl.BlockSpec((_RB, 1), lambda i: (i, 0)),
            pl.BlockSpec((1, D), lambda i: (0, 0)),
            pl.BlockSpec((D, D), lambda i: (0, 0)),
            pl.BlockSpec((1, D), lambda i: (0, 0)),
        ],
        out_specs=pl.BlockSpec((_RB, D), lambda i: (i, 0)),
        out_shape=jax.ShapeDtypeStruct((N_PAD, D), jnp.float32),
    )(aa, ab, ya, yb, dinv, b, w, bp)


@jax.jit
def kernel(x, edge_index, W0, b0, W1, b1, Wp, bp):
    row = edge_index[0].astype(jnp.int32)
    col = edge_index[1].astype(jnp.int32)
    row_p = jnp.concatenate([row, jnp.zeros((E_PAD - E,), jnp.int32)])
    col_p = jnp.concatenate([col, jnp.full((E_PAD - E,), N, jnp.int32)])
    row2d = row_p.reshape(EGROUPS, G)
    col2d = col_p.reshape(EGROUPS, G)
    x_p = jnp.pad(x, ((0, N_PAD - N), (0, 0)))

    cnt0, cnt1 = _deg(col2d)
    dinv = lax.rsqrt(cnt0 + cnt1 + 2.0).reshape(N_PAD, 1)

    ya, yb = _pre(x_p, W0, dinv)
    aa, ab = _scatter(row2d, col2d, ya, yb)
    ya, yb = _mid(aa, ab, ya, yb, dinv, b0.reshape(1, D), W1)
    aa, ab = _scatter(row2d, col2d, ya, yb)
    out = _fin(aa, ab, ya, yb, dinv, b1.reshape(1, D), Wp, bp.reshape(1, D))
    return out[:N]


# speedup vs baseline: 1.0071x; 1.0071x over previous
import functools

import jax
import jax.numpy as jnp
from jax import lax
from jax.experimental import pallas as pl
from jax.experimental.pallas import tpu as pltpu
from jax.experimental.pallas import tpu_sc as plsc

N = 10000
D = 256
H = 128
E = 160000
N_PAD = 10240
E_PAD = 163840
G = 128
ROWS_PER_TILE = N_PAD // 16
EGROUPS = E_PAD // G
EG_PER_TILE = EGROUPS // 16
EG_PER_WORKER = EGROUPS // 32
G2 = 64
EG2_PER_TILE = E_PAD // G2 // 16

_mesh = plsc.VectorSubcoreMesh(core_axis_name="c", subcore_axis_name="s")


@functools.partial(
    pl.kernel,
    out_type=(
        jax.ShapeDtypeStruct((N_PAD,), jnp.float32),
        jax.ShapeDtypeStruct((N_PAD,), jnp.float32),
    ),
    mesh=_mesh,
    scratch_types=[
        pltpu.VMEM((EG_PER_WORKER, G), jnp.int32),
        pltpu.VMEM((G,), jnp.float32),
        pltpu.VMEM((ROWS_PER_TILE,), jnp.float32),
        pltpu.VMEM_SHARED((N_PAD,), jnp.float32),
    ],
)
def _deg(col_hbm, out0, out1, cidx, ones, zbuf, hist):
    c = lax.axis_index("c")
    s = lax.axis_index("s")

    def fill(i, carry):
        ones[pl.ds(i * 16, 16)] = jnp.full((16,), 1.0, jnp.float32)
        return carry

    lax.fori_loop(0, G // 16, fill, 0)

    def zfill(i, carry):
        zbuf[pl.ds(i * 16, 16)] = jnp.zeros((16,), jnp.float32)
        return carry

    lax.fori_loop(0, ROWS_PER_TILE // 16, zfill, 0)
    pltpu.sync_copy(zbuf, hist.at[pl.ds(s * ROWS_PER_TILE, ROWS_PER_TILE)])
    plsc.subcore_barrier()

    base = c * (EGROUPS // 2) + s * EG_PER_WORKER
    pltpu.sync_copy(col_hbm.at[pl.ds(base, EG_PER_WORKER)], cidx)

    def body(j, carry):
        pltpu.sync_copy(ones, hist.at[cidx.at[j]], add=True)
        return carry

    lax.fori_loop(0, EG_PER_WORKER, body, 0)
    plsc.subcore_barrier()

    sl = pl.ds(s * ROWS_PER_TILE, ROWS_PER_TILE)

    @pl.when(c == 0)
    def _():
        pltpu.sync_copy(hist.at[sl], out0.at[sl])

    @pl.when(c == 1)
    def _():
        pltpu.sync_copy(hist.at[sl], out1.at[sl])


@functools.partial(
    pl.kernel,
    out_type=(
        jax.ShapeDtypeStruct((N_PAD, H), jnp.float32),
        jax.ShapeDtypeStruct((N_PAD, H), jnp.float32),
    ),
    mesh=_mesh,
    scratch_types=[
        pltpu.VMEM((EG2_PER_TILE // 4, G2), jnp.int32),
        pltpu.VMEM((EG2_PER_TILE // 4, G2), jnp.int32),
        pltpu.VMEM((G2, H), jnp.float32),
        pltpu.VMEM((G2, H), jnp.float32),
        pltpu.VMEM((G2, H), jnp.float32),
        pltpu.VMEM((G2, H), jnp.float32),
        pltpu.VMEM_SHARED((N_PAD, H), jnp.float32),
        pltpu.SemaphoreType.DMA,
        pltpu.SemaphoreType.DMA,
        pltpu.SemaphoreType.DMA,
        pltpu.SemaphoreType.DMA,
    ],
)
def _scatter(row_hbm, col_hbm, ya_hbm, yb_hbm, outa, outb, ridx, cidx,
             bufa, bufb, bufc, bufd, acc, sema, semb, semc, semd):
    c = lax.axis_index("c")
    s = lax.axis_index("s")
    bufs = (bufa, bufb, bufc, bufd)
    sems = (sema, semb, semc, semd)

    def zrow(i, carry):
        for k in range(H // 16):
            bufa[i, pl.ds(k * 16, 16)] = jnp.zeros((16,), jnp.float32)
            bufb[i, pl.ds(k * 16, 16)] = jnp.zeros((16,), jnp.float32)
        return carry

    lax.fori_loop(0, G2, zrow, 0)
    for t in range(ROWS_PER_TILE // (2 * G2)):
        pltpu.sync_copy(bufa, acc.at[pl.ds(s * ROWS_PER_TILE + 2 * t * G2, G2)])
        pltpu.sync_copy(bufb, acc.at[pl.ds(s * ROWS_PER_TILE + (2 * t + 1) * G2, G2)])
    plsc.subcore_barrier()

    PH = EG2_PER_TILE // 4

    def make_body(y_hbm):
        def gstart(j, buf, sem):
            jw = jnp.where(j < PH, j, 0)
            pltpu.async_copy(y_hbm.at[ridx.at[jw]], buf, sem)

        def gwait(buf, sem):
            pltpu.make_async_copy(y_hbm.at[ridx.at[0]], buf, sem).wait()

        def run():
            for phase in range(4):
                ebase = s * EG2_PER_TILE + phase * PH
                pltpu.sync_copy(row_hbm.at[pl.ds(ebase, PH)], ridx)
                pltpu.sync_copy(col_hbm.at[pl.ds(ebase, PH)], cidx)
                for u in range(4):
                    gstart(jnp.int32(u), bufs[u], sems[u])

                def body(k, carry):
                    j = k * 4
                    for u in range(4):
                        gwait(bufs[u], sems[u])
                        pltpu.sync_copy(bufs[u], acc.at[cidx.at[j + u]],
                                        add=True)
                        gstart(j + u + 4, bufs[u], sems[u])
                    return carry

                lax.fori_loop(0, PH // 4, body, 0)
                for u in range(4):
                    gwait(bufs[u], sems[u])

        return run

    @pl.when(c == 0)
    def _():
        make_body(ya_hbm)()

    @pl.when(c == 1)
    def _():
        make_body(yb_hbm)()

    plsc.subcore_barrier()

    sl = pl.ds(s * ROWS_PER_TILE, ROWS_PER_TILE)

    @pl.when(c == 0)
    def _():
        pltpu.sync_copy(acc.at[sl], outa.at[sl])

    @pl.when(c == 1)
    def _():
        pltpu.sync_copy(acc.at[sl], outb.at[sl])


_RB = 1024
_GRID = N_PAD // _RB


def _gelu(v):
    return 0.5 * v * (1.0 + lax.erf(v * 0.7071067811865476))


def _pre_body(x_ref, w_ref, dinv_ref, ya_ref, yb_ref):
    y = jnp.dot(x_ref[...], w_ref[...], preferred_element_type=jnp.float32)
    y = y * dinv_ref[...]
    ya_ref[...] = y[:, :H]
    yb_ref[...] = y[:, H:]


def _pre(x, w, dinv):
    return pl.pallas_call(
        _pre_body,
        grid=(_GRID,),
        in_specs=[
            pl.BlockSpec((_RB, D), lambda i: (i, 0)),
            pl.BlockSpec((D, D), lambda i: (0, 0)),
            pl.BlockSpec((_RB, 1), lambda i: (i, 0)),
        ],
        out_specs=(
            pl.BlockSpec((_RB, H), lambda i: (i, 0)),
            pl.BlockSpec((_RB, H), lambda i: (i, 0)),
        ),
        out_shape=(
            jax.ShapeDtypeStruct((N_PAD, H), jnp.float32),
            jax.ShapeDtypeStruct((N_PAD, H), jnp.float32),
        ),
    )(x, w, dinv)


def _mid_body(aa_ref, ab_ref, ya_ref, yb_ref, dinv_ref, b_ref, w_ref,
              oa_ref, ob_ref):
    dinv = dinv_ref[...]
    ha = dinv * (aa_ref[...] + 2.0 * ya_ref[...])
    hb = dinv * (ab_ref[...] + 2.0 * yb_ref[...])
    h = jnp.concatenate([ha, hb], axis=1) + b_ref[...]
    h = _gelu(h)
    y = jnp.dot(h, w_ref[...], preferred_element_type=jnp.float32) * dinv
    oa_ref[...] = y[:, :H]
    ob_ref[...] = y[:, H:]


def _mid(aa, ab, ya, yb, dinv, b, w):
    return pl.pallas_call(
        _mid_body,
        grid=(_GRID,),
        in_specs=[
            pl.BlockSpec((_RB, H), lambda i: (i, 0)),
            pl.BlockSpec((_RB, H), lambda i: (i, 0)),
            pl.BlockSpec((_RB, H), lambda i: (i, 0)),
            pl.BlockSpec((_RB, H), lambda i: (i, 0)),
            pl.BlockSpec((_RB, 1), lambda i: (i, 0)),
            pl.BlockSpec((1, D), lambda i: (0, 0)),
            pl.BlockSpec((D, D), lambda i: (0, 0)),
        ],
        out_specs=(
            pl.BlockSpec((_RB, H), lambda i: (i, 0)),
            pl.BlockSpec((_RB, H), lambda i: (i, 0)),
        ),
        out_shape=(
            jax.ShapeDtypeStruct((N_PAD, H), jnp.float32),
            jax.ShapeDtypeStruct((N_PAD, H), jnp.float32),
        ),
    )(aa, ab, ya, yb, dinv, b, w)


def _fin_body(aa_ref, ab_ref, ya_ref, yb_ref, dinv_ref, b_ref, w_ref,
              bp_ref, o_ref):
    dinv = dinv_ref[...]
    ha = dinv * (aa_ref[...] + 2.0 * ya_ref[...])
    hb = dinv * (ab_ref[...] + 2.0 * yb_ref[...])
    h = jnp.concatenate([ha, hb], axis=1) + b_ref[...]
    h = _gelu(h)
    o_ref[...] = (
        jnp.dot(h, w_ref[...], preferred_element_type=jnp.float32) + bp_ref[...]
    )


def _fin(aa, ab, ya, yb, dinv, b, w, bp):
    return pl.pallas_call(
        _fin_body,
        grid=(_GRID,),
        in_specs=[
            pl.BlockSpec((_RB, H), lambda i: (i, 0)),
            pl.BlockSpec((_RB, H), lambda i: (i, 0)),
            pl.BlockSpec((_RB, H), lambda i: (i, 0)),
            pl.BlockSpec((_RB, H), lambda i: (i, 0)),
            pl.BlockSpec((_RB, 1), lambda i: (i, 0)),
            pl.BlockSpec((1, D), lambda i: (0, 0)),
            pl.BlockSpec((D, D), lambda i: (0, 0)),
            pl.BlockSpec((1, D), lambda i: (0, 0)),
        ],
        out_specs=pl.BlockSpec((_RB, D), lambda i: (i, 0)),
        out_shape=jax.ShapeDtypeStruct((N_PAD, D), jnp.float32),
    )(aa, ab, ya, yb, dinv, b, w, bp)


@jax.jit
def kernel(x, edge_index, W0, b0, W1, b1, Wp, bp):
    row = edge_index[0].astype(jnp.int32)
    col = edge_index[1].astype(jnp.int32)
    row_p = jnp.concatenate([row, jnp.zeros((E_PAD - E,), jnp.int32)])
    col_p = jnp.concatenate([col, jnp.full((E_PAD - E,), N, jnp.int32)])
    row2d = row_p.reshape(EGROUPS, G)
    col2d = col_p.reshape(EGROUPS, G)
    x_p = jnp.pad(x, ((0, N_PAD - N), (0, 0)))

    cnt0, cnt1 = _deg(col2d)
    dinv = lax.rsqrt(cnt0 + cnt1 + 2.0).reshape(N_PAD, 1)

    row2d_s = row_p.reshape(E_PAD // G2, G2)
    col2d_s = col_p.reshape(E_PAD // G2, G2)
    ya, yb = _pre(x_p, W0, dinv)
    aa, ab = _scatter(row2d_s, col2d_s, ya, yb)
    ya, yb = _mid(aa, ab, ya, yb, dinv, b0.reshape(1, D), W1)
    aa, ab = _scatter(row2d_s, col2d_s, ya, yb)
    out = _fin(aa, ab, ya, yb, dinv, b1.reshape(1, D), Wp, bp.reshape(1, D))
    return out[:N]
